# Initial kernel scaffold; baseline (speedup 1.0000x reference)
import jax
import jax.numpy as jnp
from jax.experimental import pallas as pl


def kernel(x, edge_index, batch, params):
    raise NotImplementedError("write your pallas kernel here")


# speedup vs baseline: 5.6735x; 5.6735x over previous
import functools

import jax
import jax.numpy as jnp
from jax import lax
from jax.experimental import pallas as pl
from jax.experimental.pallas import tpu as pltpu
from jax.experimental.pallas import tpu_sc as plsc

N = 10000
E = 320000
D = 128
NG = 64
NC = 2
NS = 16
NW = NC * NS
EPW = E // NW
CH = 80
K = EPW // CH
NP = 10240
RPS = NP // NS
ZR = 16

@functools.cache
def _mesh():
    return plsc.VectorSubcoreMesh(core_axis_name="c", subcore_axis_name="s")


def _agg_sc(h, src3, dst3):
    @functools.partial(
        pl.kernel,
        out_type=jax.ShapeDtypeStruct((NC, NP, D), jnp.float32),
        mesh=_mesh(),
        scratch_types=[
            pltpu.VMEM((K, CH), jnp.int32),
            pltpu.VMEM((K, CH), jnp.int32),
            pltpu.VMEM((CH, D), jnp.float32),
            pltpu.VMEM((ZR, D), jnp.float32),
            pltpu.VMEM_SHARED((NP, D), jnp.float32),
            pltpu.SemaphoreType.DMA,
        ],
    )
    def k(h_hbm, src_hbm, dst_hbm, out_hbm, src_v, dst_v, rows_v, z_v, acc_sh, sem):
        cid = lax.axis_index("c")
        sid = lax.axis_index("s")
        wid = cid * NS + sid

        zv = jnp.zeros((16,), jnp.float32)

        @pl.loop(0, ZR)
        def _(r):
            @pl.loop(0, D, step=16)
            def _(cc):
                z_v.at[r, pl.ds(cc, 16)][...] = zv

        @pl.loop(0, RPS, step=ZR)
        def _(r0):
            pltpu.sync_copy(z_v, acc_sh.at[pl.ds(sid * RPS + r0, ZR)])

        pltpu.sync_copy(src_hbm.at[wid], src_v)
        pltpu.sync_copy(dst_hbm.at[wid], dst_v)
        plsc.subcore_barrier()

        @pl.loop(0, K)
        def _(j):
            pltpu.async_copy(h_hbm.at[src_v.at[j]], rows_v, sem).wait()
            pltpu.sync_copy(rows_v, acc_sh.at[dst_v.at[j]], add=True)

        plsc.subcore_barrier()
        pltpu.sync_copy(
            acc_sh.at[pl.ds(sid * RPS, RPS)],
            out_hbm.at[cid, pl.ds(sid * RPS, RPS)],
        )

    return k(h, src3, dst3)


def _xla_mean_ref(zr, r, c):
    hs = r // 16

    def half(base):
        def step(i, acc):
            return acc + zr[pl.ds(base + i * 8, 8), :]
        acc = lax.fori_loop(0, hs, step, jnp.zeros((8, c), jnp.float32))
        t = acc[0:4] + acc[4:8]
        t = t[0:2] + t[2:4]
        return t[0:1] + t[1:2]

    return (half(0) + half(hs * 8)) * jnp.float32(1.0 / r)


def _bn_relu(z, g, be, scratch):
    r, c = z.shape
    scratch[...] = z
    m = _xla_mean_ref(scratch, r, c)
    cen = z - m
    scratch[...] = cen * cen
    v = _xla_mean_ref(scratch, r, c)
    return jnp.maximum((z - m) / jnp.sqrt(v + 1e-5) * g + be, 0.0)


def _mlp_tc(h, agg, w1, b1, g1, be1, w2, b2, g2, be2):
    def body(h_ref, a_ref, w1_ref, b1_ref, g1_ref, be1_ref,
             w2_ref, b2_ref, g2_ref, be2_ref, o_ref, s1_ref, s2_ref):
        u = h_ref[...] + (a_ref[0] + a_ref[1])
        z = jnp.dot(u, w1_ref[...], preferred_element_type=jnp.float32) + b1_ref[...]
        z = _bn_relu(z, g1_ref[...], be1_ref[...], s1_ref)
        z = jnp.dot(z, w2_ref[...], preferred_element_type=jnp.float32) + b2_ref[...]
        o_ref[...] = _bn_relu(z, g2_ref[...], be2_ref[...], s2_ref)

    return pl.pallas_call(
        body,
        out_shape=jax.ShapeDtypeStruct((N, D), jnp.float32),
        scratch_shapes=[pltpu.VMEM((N, 2 * D), jnp.float32),
                        pltpu.VMEM((N, D), jnp.float32)],
    )(h, agg[:, :N, :], w1, b1.reshape(1, -1), g1.reshape(1, -1), be1.reshape(1, -1),
      w2, b2.reshape(1, -1), g2.reshape(1, -1), be2.reshape(1, -1))


def _head_tc(h, batch_row, w, b, g, be, cw, cb):
    def body(h_ref, bt_ref, w_ref, b_ref, g_ref, be_ref, cw_ref, cb_ref, o_ref,
             s_ref):
        ids = bt_ref[...]
        onehot = (lax.broadcasted_iota(jnp.int32, (NG, N), 0) == ids).astype(jnp.float32)
        pooled = jnp.dot(onehot, h_ref[...], preferred_element_type=jnp.float32,
                         precision=lax.Precision.HIGHEST)
        z = jnp.dot(pooled, w_ref[...], preferred_element_type=jnp.float32) + b_ref[...]
        z = _bn_relu(z, g_ref[...], be_ref[...], s_ref)
        o_ref[...] = jnp.dot(z, cw_ref[...], preferred_element_type=jnp.float32) + cb_ref[...]

    return pl.pallas_call(
        body,
        out_shape=jax.ShapeDtypeStruct((NG, 1), jnp.float32),
        scratch_shapes=[pltpu.VMEM((NG, D), jnp.float32)],
    )(h, batch_row, w, b.reshape(1, -1), g.reshape(1, -1), be.reshape(1, -1),
      cw, cb.reshape(1, -1))


def kernel(x, edge_index, batch, params):
    src3 = edge_index[0].reshape(NW, K, CH)
    dst3 = edge_index[1].reshape(NW, K, CH)
    batch_row = batch.reshape(1, N)
    h = x
    for i in range(3):
        p = params[f"layer{i}"]
        agg = _agg_sc(h, src3, dst3)
        h = _mlp_tc(h, agg, p["W1"], p["b1"], p["g1"], p["be1"],
                    p["W2"], p["b2"], p["g2"], p["be2"])
    out = _head_tc(h, batch_row, params["lin1_W"], params["lin1_b"],
                   params["bn1_g"], params["bn1_b"],
                   params["cls_W"], params["cls_b"])
    return out.reshape(-1)
